# baseline (speedup 1.0000x reference)
import jax
import jax.numpy as jnp
from jax.experimental import pallas as pl

_B, _N, _M = 16, 2048, 64


_RS = 16


def _body(dist_ref, pt_ref, types_ref, coef_ref, out_ref, epa_ref):
    b = pl.program_id(0)

    def strip(k, carry):
        ph_acc, s_acc = carry
        row = k * _RS
        d = dist_ref[0, pl.ds(row, _RS), :]
        pt = pt_ref[0, pl.ds(row, _RS), :]
        is1 = pt == 1
        is2 = pt == 2

        def sel(i):
            return jnp.where(is1, coef_ref[i, 1],
                             jnp.where(is2, coef_ref[i, 2], coef_ref[i, 0]))

        x = jnp.clip(sel(5) * d - sel(4), 0.0, 1.0)
        x3 = x * x * x
        fc = 1.0 - x3 * (x * (6.0 * x - 15.0) + 10.0)

        half_phi = jnp.exp2(sel(0) - sel(1) * d) * fc
        rho_e = jnp.exp2(sel(2) - sel(3) * d) * fc

        return (ph_acc + jnp.sum(half_phi),
                s_acc + jnp.sum(rho_e, axis=0, keepdims=True))

    half_phi_sum, s = jax.lax.fori_loop(
        0, _M // _RS, strip,
        (jnp.float32(0.0), jnp.zeros((1, _N), jnp.float32)))
    s = s + 1e-12
    emb_sum = jnp.sum(s * jax.lax.rsqrt(s))

    tt = types_ref[pl.ds(b, 1), :]
    off_sum = jnp.sum(jnp.where(tt == 1, coef_ref[6, 1], coef_ref[6, 0]))

    e = jnp.reshape(half_phi_sum - emb_sum + off_sum, (1, 1))
    out_ref[pl.ds(b, 1), :] = e
    epa_ref[pl.ds(b, 1), :] = e * (1.0 / _N)


def kernel(types, pair_types, distances, A, xi, p, q, r0, offset, cut_a, cut_b):
    dist_t = distances.transpose(0, 2, 1)
    pt_t = pair_types.transpose(0, 2, 1)

    inv_ln2 = 1.4426950408889634
    inv_ba = 1.0 / (cut_b - cut_a)
    coef = jnp.concatenate([
        jnp.stack([
            jnp.log2(0.5 * A) + p * inv_ln2,
            (p / r0) * inv_ln2,
            2.0 * jnp.log2(xi) + 2.0 * q * inv_ln2,
            (2.0 * q / r0) * inv_ln2,
            cut_a * inv_ba,
            inv_ba,
        ]),
        jnp.pad(offset, (0, 1)).reshape(1, 3),
    ])

    energy, energy_per_atom = pl.pallas_call(
        _body,
        grid=(_B,),
        in_specs=[
            pl.BlockSpec((1, _M, _N), lambda b: (b, 0, 0)),
            pl.BlockSpec((1, _M, _N), lambda b: (b, 0, 0)),
            pl.BlockSpec((_B, _N), lambda b: (0, 0)),
            pl.BlockSpec((7, 3), lambda b: (0, 0)),
        ],
        out_specs=[
            pl.BlockSpec((_B, 1), lambda b: (0, 0)),
            pl.BlockSpec((_B, 1), lambda b: (0, 0)),
        ],
        out_shape=[
            jax.ShapeDtypeStruct((_B, 1), jnp.float32),
            jax.ShapeDtypeStruct((_B, 1), jnp.float32),
        ],
    )(dist_t, pt_t, types, coef)

    return (energy, energy_per_atom)


# speedup vs baseline: 1.2480x; 1.2480x over previous
import jax
import jax.numpy as jnp
from jax.experimental import pallas as pl
from jax.experimental.pallas import tpu as pltpu

_B, _N, _M = 16, 2048, 64


def _body(dist_ref, pt_ref, types_ref, coef_ref, out_ref, epa_ref):
    b = pl.program_id(0)
    d = dist_ref[0]
    pt = pt_ref[0]
    is1 = pt == 1
    is2 = pt == 2

    def sel(i):
        return jnp.where(is1, coef_ref[i, 1],
                         jnp.where(is2, coef_ref[i, 2], coef_ref[i, 0]))

    x = jnp.clip(sel(5) * d - sel(4), 0.0, 1.0)
    x3 = x * x * x
    fc = 1.0 - x3 * (x * (6.0 * x - 15.0) + 10.0)

    half_phi = jnp.exp2(sel(0) - sel(1) * d) * fc
    rho_e = jnp.exp2(sel(2) - sel(3) * d) * fc

    half_phi_sum = jnp.sum(half_phi)
    s = jnp.sum(rho_e, axis=0, keepdims=True) + 1e-12
    emb_sum = jnp.sum(s * jax.lax.rsqrt(s))

    tt = types_ref[pl.ds(b, 1), :]
    off_sum = jnp.sum(jnp.where(tt == 1, coef_ref[6, 1], coef_ref[6, 0]))

    e = half_phi_sum - emb_sum + off_sum
    out_ref[0] = jnp.full((1, 128), e, jnp.float32)
    epa_ref[0] = jnp.full((1, 128), e * (1.0 / _N), jnp.float32)


def kernel(types, pair_types, distances, A, xi, p, q, r0, offset, cut_a, cut_b):
    dist_t = distances.transpose(0, 2, 1)
    pt_t = pair_types.transpose(0, 2, 1)

    inv_ln2 = 1.4426950408889634
    inv_ba = 1.0 / (cut_b - cut_a)
    coef = jnp.concatenate([
        jnp.stack([
            jnp.log2(0.5 * A) + p * inv_ln2,
            (p / r0) * inv_ln2,
            2.0 * jnp.log2(xi) + 2.0 * q * inv_ln2,
            (2.0 * q / r0) * inv_ln2,
            cut_a * inv_ba,
            inv_ba,
        ]),
        jnp.pad(offset, (0, 1)).reshape(1, 3),
    ])

    energy, energy_per_atom = pl.pallas_call(
        _body,
        compiler_params=pltpu.CompilerParams(
            dimension_semantics=("parallel",)),
        grid=(_B,),
        in_specs=[
            pl.BlockSpec((1, _M, _N), lambda b: (b, 0, 0)),
            pl.BlockSpec((1, _M, _N), lambda b: (b, 0, 0)),
            pl.BlockSpec((_B, _N), lambda b: (0, 0)),
            pl.BlockSpec((7, 3), lambda b: (0, 0)),
        ],
        out_specs=[
            pl.BlockSpec((1, 1, 128), lambda b: (b, 0, 0)),
            pl.BlockSpec((1, 1, 128), lambda b: (b, 0, 0)),
        ],
        out_shape=[
            jax.ShapeDtypeStruct((_B, 1, 128), jnp.float32),
            jax.ShapeDtypeStruct((_B, 1, 128), jnp.float32),
        ],
    )(dist_t, pt_t, types, coef)

    return (energy[:, 0, :1], energy_per_atom[:, 0, :1])
